# Initial kernel scaffold; baseline (speedup 1.0000x reference)
import functools

import jax
import jax.numpy as jnp
from jax import lax
from jax.experimental import pallas as pl
from jax.experimental.pallas import tpu as pltpu
from jax.experimental.pallas import tpu_sc as plsc

VOCAB = 1000
N_TOK = 51200
NC, NS, LANES = 2, 16, 16
NW = NC * NS
PER_W = N_TOK // NW
CH = 32
NCH = PER_W // CH


def _logz_body(table_ref, out_ref):
    x = table_ref[...]
    m = jnp.max(x, axis=1, keepdims=True)
    s = jnp.sum(jnp.exp(x - m), axis=1, keepdims=True)
    out_ref[...] = m + jnp.log(s)


def _logz_call(table):
    return pl.pallas_call(
        _logz_body,
        out_shape=jax.ShapeDtypeStruct((VOCAB, 1), jnp.float32),
    )(table)


_mesh = plsc.VectorSubcoreMesh(core_axis_name="c", subcore_axis_name="s")


@functools.partial(
    pl.kernel,
    out_type=[
        jax.ShapeDtypeStruct((N_TOK, VOCAB), jnp.float32),
        jax.ShapeDtypeStruct((NW, LANES), jnp.float32),
    ],
    mesh=_mesh,
    scratch_types=[
        pltpu.VMEM((NCH, CH), jnp.int32),
        pltpu.VMEM((NCH, CH), jnp.int32),
        pltpu.VMEM((CH, VOCAB), jnp.float32),
        pltpu.VMEM((CH,), jnp.float32),
        pltpu.VMEM((LANES,), jnp.float32),
        pltpu.SemaphoreType.DMA,
        pltpu.SemaphoreType.DMA,
    ],
)
def _sc_main(idx_hbm, tgt_hbm, table_hbm, logz_hbm, out_hbm, part_hbm,
             idx_v, tgt_v, rows_v, lz_v, acc_v, sem_r, sem_z):
    wid = lax.axis_index("s") * NC + lax.axis_index("c")
    base = wid * PER_W
    pltpu.sync_copy(idx_hbm.at[wid], idx_v)
    pltpu.sync_copy(tgt_hbm.at[wid], tgt_v)
    lane = lax.iota(jnp.int32, LANES)

    def chunk(g, acc):
        idx_row = idx_v.at[g]
        cp_rows = pltpu.async_copy(table_hbm.at[idx_row], rows_v, sem_r)
        cp_lz = pltpu.async_copy(logz_hbm.at[idx_row], lz_v, sem_z)
        cp_rows.wait()
        pltpu.sync_copy(rows_v, out_hbm.at[pl.ds(base + g * CH, CH)])
        cp_lz.wait()
        for s in range(CH // LANES):
            sl = pl.ds(s * LANES, LANES)
            tcol = tgt_v[g, sl]
            ll = plsc.load_gather(rows_v, [lane + s * LANES, tcol])
            acc = acc + (lz_v[sl] - ll)
        return acc

    acc = lax.fori_loop(0, NCH, chunk, jnp.zeros((LANES,), jnp.float32))
    acc_v[...] = acc
    pltpu.sync_copy(acc_v, part_hbm.at[wid])


def _loss_body(p_ref, out_ref):
    out_ref[...] = (jnp.sum(p_ref[...]) / N_TOK).reshape(1, 1)


def _loss_call(parts):
    return pl.pallas_call(
        _loss_body,
        out_shape=jax.ShapeDtypeStruct((1, 1), jnp.float32),
    )(parts)


def kernel(idx, targets, table):
    idx3 = idx.reshape(NW, NCH, CH).astype(jnp.int32)
    tgt3 = targets.reshape(NW, NCH, CH).astype(jnp.int32)
    logz = _logz_call(table).reshape(VOCAB)
    logits2d, parts = _sc_main(idx3, tgt3, table, logz)
    loss = _loss_call(parts).reshape(())
    return (logits2d, loss)


# speedup vs baseline: 1.3839x; 1.3839x over previous
import functools

import jax
import jax.numpy as jnp
from jax import lax
from jax.experimental import pallas as pl
from jax.experimental.pallas import tpu as pltpu
from jax.experimental.pallas import tpu_sc as plsc

VOCAB = 1000
N_TOK = 51200
NC, NS, LANES = 2, 16, 16
NW = NC * NS
PER_W = N_TOK // NW
CH = 32
NCH = PER_W // CH


def _logz_body(table_ref, out_ref):
    x = table_ref[...]
    m = jnp.max(x, axis=1, keepdims=True)
    s = jnp.sum(jnp.exp(x - m), axis=1, keepdims=True)
    out_ref[...] = m + jnp.log(s)


def _logz_call(table):
    return pl.pallas_call(
        _logz_body,
        out_shape=jax.ShapeDtypeStruct((VOCAB, 1), jnp.float32),
    )(table)


_mesh = plsc.VectorSubcoreMesh(core_axis_name="c", subcore_axis_name="s")


@functools.partial(
    pl.kernel,
    out_type=[
        jax.ShapeDtypeStruct((N_TOK, VOCAB), jnp.float32),
        jax.ShapeDtypeStruct((NW, LANES), jnp.float32),
    ],
    mesh=_mesh,
    compiler_params=pltpu.CompilerParams(use_tc_tiling_on_sc=False),
    scratch_types=[
        pltpu.VMEM((NCH, CH), jnp.int32),
        pltpu.VMEM((NCH, CH), jnp.int32),
        pltpu.VMEM((CH, VOCAB), jnp.float32),
        pltpu.VMEM((CH, VOCAB), jnp.float32),
        pltpu.VMEM((2, CH), jnp.int32),
        pltpu.VMEM((2, CH), jnp.float32),
        pltpu.VMEM((2, CH), jnp.float32),
        pltpu.VMEM((LANES,), jnp.float32),
        pltpu.SemaphoreType.DMA,
        pltpu.SemaphoreType.DMA,
        pltpu.SemaphoreType.DMA,
        pltpu.SemaphoreType.DMA,
        pltpu.SemaphoreType.DMA,
        pltpu.SemaphoreType.DMA,
        pltpu.SemaphoreType.DMA,
        pltpu.SemaphoreType.DMA,
    ],
)
def _sc_main(idx_hbm, tgt_hbm, table_hbm, tflat_hbm, logz_hbm, out_hbm,
             part_hbm, idx_v, tgt_v, rows0, rows1, pos_v, lz_v, ll_v, acc_v,
             sem_r0, sem_r1, sem_w0, sem_w1, sem_z0, sem_z1, sem_l0, sem_l1):
    wid = lax.axis_index("s") * NC + lax.axis_index("c")
    base = wid * PER_W
    pltpu.sync_copy(idx_hbm.at[wid], idx_v)
    pltpu.sync_copy(tgt_hbm.at[wid], tgt_v)

    rows = (rows0, rows1)
    sem_r = (sem_r0, sem_r1)
    sem_w = (sem_w0, sem_w1)
    sem_z = (sem_z0, sem_z1)
    sem_l = (sem_l0, sem_l1)

    def issue_chunk(k, b):
        idx_row = idx_v.at[k]
        pltpu.async_copy(table_hbm.at[idx_row], rows[b], sem_r[b])
        for s in range(CH // LANES):
            sl = pl.ds(s * LANES, LANES)
            pos_v[b, sl] = idx_v[k, sl] * VOCAB + tgt_v[k, sl]
        pltpu.async_copy(logz_hbm.at[idx_row], lz_v.at[b], sem_z[b])
        pltpu.async_copy(tflat_hbm.at[pos_v.at[b]], ll_v.at[b], sem_l[b])

    def wait_gather(b):
        pltpu.make_async_copy(table_hbm.at[idx_v.at[0]], rows[b],
                              sem_r[b]).wait()

    def issue_scatter(k, b):
        pltpu.async_copy(rows[b], out_hbm.at[pl.ds(base + k * CH, CH)],
                         sem_w[b])

    def wait_scatter(b):
        pltpu.make_async_copy(rows[b], out_hbm.at[pl.ds(base, CH)],
                              sem_w[b]).wait()

    def loss_math(k, b, acc):
        pltpu.make_async_copy(logz_hbm.at[idx_v.at[0]], lz_v.at[b],
                              sem_z[b]).wait()
        pltpu.make_async_copy(tflat_hbm.at[pos_v.at[b]], ll_v.at[b],
                              sem_l[b]).wait()
        for s in range(CH // LANES):
            sl = pl.ds(s * LANES, LANES)
            acc = acc + (lz_v[b, sl] - ll_v[b, sl])
        return acc

    issue_chunk(0, 0)
    wait_gather(0)
    issue_scatter(0, 0)
    issue_chunk(1, 1)
    acc = loss_math(0, 0, jnp.zeros((LANES,), jnp.float32))

    def pair(j, acc):
        for t in range(2):
            k = 2 * j + 1 + t
            b = 1 - t
            wait_gather(b)
            issue_scatter(k, b)
            wait_scatter(1 - b)
            issue_chunk(k + 1, 1 - b)
            acc = loss_math(k, b, acc)
        return acc

    acc = lax.fori_loop(0, (NCH - 2) // 2, pair, acc)

    wait_gather(1)
    issue_scatter(NCH - 1, 1)
    acc = loss_math(NCH - 1, 1, acc)
    acc_v[...] = acc
    pltpu.sync_copy(acc_v, part_hbm.at[wid])
    wait_scatter(0)
    wait_scatter(1)


def _loss_body(p_ref, out_ref):
    out_ref[...] = (jnp.sum(p_ref[...]) / N_TOK).reshape(1, 1)


def _loss_call(parts):
    return pl.pallas_call(
        _loss_body,
        out_shape=jax.ShapeDtypeStruct((1, 1), jnp.float32),
    )(parts)


def kernel(idx, targets, table):
    idx3 = idx.reshape(NW, NCH, CH).astype(jnp.int32)
    tgt3 = targets.reshape(NW, NCH, CH).astype(jnp.int32)
    logz = _logz_call(table).reshape(VOCAB)
    tflat = jnp.pad(table.reshape(-1), (0, 8))
    logits2d, parts = _sc_main(idx3, tgt3, table, tflat, logz)
    loss = _loss_call(parts).reshape(())
    return (logits2d, loss)
